# initial kernel scaffold (unmeasured)
import jax
import jax.numpy as jnp
from jax import lax
from jax.experimental import pallas as pl
from jax.experimental.pallas import tpu as pltpu


def kernel(
    x,
):
    def body(*refs):
        pass

    out_shape = jax.ShapeDtypeStruct(..., jnp.float32)
    return pl.pallas_call(body, out_shape=out_shape)(...)


# baseline (device time: 160710 ns/iter reference)
import jax
import jax.numpy as jnp
from jax import lax
from jax.experimental import pallas as pl
from jax.experimental.pallas import tpu as pltpu

P = 4


def kernel(x):
    m, n = x.shape
    chunk = m // P

    def body(x_ref, out_ref, recv_buf, send_sems, recv_sems):
        my_x = lax.axis_index("x")
        my_y = lax.axis_index("y")
        my_z = lax.axis_index("z")
        right = (my_y + 1) % P
        left = (my_y - 1) % P

        barrier_sem = pltpu.get_barrier_semaphore()
        for nbr in (left, right):
            pl.semaphore_signal(
                barrier_sem,
                inc=1,
                device_id=(my_x, nbr, my_z),
                device_id_type=pl.DeviceIdType.MESH,
            )
        pl.semaphore_wait(barrier_sem, 2)

        out_ref[...] = x_ref[...].astype(jnp.bfloat16)

        for h in range(P - 1):
            s = (my_y - h) % P
            r = (my_y - h - 1) % P
            rdma = pltpu.make_async_remote_copy(
                src_ref=out_ref.at[pl.ds(s * chunk, chunk), :],
                dst_ref=recv_buf.at[h],
                send_sem=send_sems.at[h],
                recv_sem=recv_sems.at[h],
                device_id=(my_x, right, my_z),
                device_id_type=pl.DeviceIdType.MESH,
            )
            rdma.start()
            rdma.wait()
            out_ref[pl.ds(r * chunk, chunk), :] = (
                out_ref[pl.ds(r * chunk, chunk), :] + recv_buf[h]
            )

        for h in range(P - 1):
            k = P - 1 + h
            s = (my_y + 1 - h) % P
            rdma = pltpu.make_async_remote_copy(
                src_ref=out_ref.at[pl.ds(s * chunk, chunk), :],
                dst_ref=out_ref.at[pl.ds(s * chunk, chunk), :],
                send_sem=send_sems.at[k],
                recv_sem=recv_sems.at[k],
                device_id=(my_x, right, my_z),
                device_id_type=pl.DeviceIdType.MESH,
            )
            rdma.start()
            rdma.wait()

    return pl.pallas_call(
        body,
        out_shape=jax.ShapeDtypeStruct((m, n), jnp.bfloat16),
        in_specs=[pl.BlockSpec(memory_space=pltpu.VMEM)],
        out_specs=pl.BlockSpec(memory_space=pltpu.VMEM),
        scratch_shapes=[
            pltpu.VMEM((P - 1, chunk, n), jnp.bfloat16),
            pltpu.SemaphoreType.DMA((2 * (P - 1),)),
            pltpu.SemaphoreType.DMA((2 * (P - 1),)),
        ],
        compiler_params=pltpu.CompilerParams(collective_id=0),
    )(x)


# device time: 111260 ns/iter; 1.4445x vs baseline; 1.4445x over previous
import jax
import jax.numpy as jnp
from jax import lax
from jax.experimental import pallas as pl
from jax.experimental.pallas import tpu as pltpu

P = 4
K = 16

PH_IN = 0
PH_MID = 1
PH_OUT = 2
PH_X = 3


def kernel(x):
    m, n = x.shape
    hn = n // 2
    rs = m // K

    def body(x_ref, out_ref, r1_buf, r2_buf, send_sems, recv_sems):
        my_x = lax.axis_index("x")
        my_y = lax.axis_index("y")
        my_z = lax.axis_index("z")

        is_end = jnp.logical_or(my_y == 0, my_y == 3)
        inner_y = jnp.where(my_y == 0, 1, 2)
        end_y = jnp.where(my_y == 1, 0, 3)
        om_y = jnp.where(my_y == 1, 2, 1)
        xp = 1 - my_x

        barrier = pltpu.get_barrier_semaphore()

        @pl.when(is_end)
        def _():
            for dev in ((my_x, inner_y, my_z), (xp, my_y, my_z)):
                pl.semaphore_signal(
                    barrier, inc=1, device_id=dev,
                    device_id_type=pl.DeviceIdType.MESH,
                )
            pl.semaphore_wait(barrier, 2)

        @pl.when(jnp.logical_not(is_end))
        def _():
            for dev in (
                (my_x, end_y, my_z),
                (my_x, om_y, my_z),
                (xp, my_y, my_z),
            ):
                pl.semaphore_signal(
                    barrier, inc=1, device_id=dev,
                    device_id_type=pl.DeviceIdType.MESH,
                )
            pl.semaphore_wait(barrier, 3)

        def mk(phase, i, src, dst, dev):
            return pltpu.make_async_remote_copy(
                src_ref=src,
                dst_ref=dst,
                send_sem=send_sems.at[phase, i],
                recv_sem=recv_sems.at[phase, i],
                device_id=dev,
                device_id_type=pl.DeviceIdType.MESH,
            )

        def end_program(c0):
            c = slice(c0, c0 + hn)
            oc = slice(hn - c0, 2 * hn - c0)
            s1 = []
            for i in range(K):
                r = pl.ds(i * rs, rs)
                out_ref[r, c] = x_ref[r, c].astype(out_ref.dtype)
                rdma = mk(PH_IN, i, out_ref.at[r, c], r1_buf.at[r],
                          (my_x, inner_y, my_z))
                rdma.start()
                s1.append(rdma)
            s4 = []
            for i in range(K):
                r = pl.ds(i * rs, rs)
                mk(PH_OUT, i, out_ref.at[r, c], out_ref.at[r, c],
                   (my_x, inner_y, my_z)).wait_recv()
                rdma = mk(PH_X, i, out_ref.at[r, c], out_ref.at[r, c],
                          (xp, my_y, my_z))
                rdma.start()
                s4.append(rdma)
            for i in range(K):
                r = pl.ds(i * rs, rs)
                mk(PH_X, i, out_ref.at[r, c], out_ref.at[r, oc],
                   (xp, my_y, my_z)).wait_recv()
            for i in range(K):
                s1[i].wait_send()
                s4[i].wait_send()

        def mid_program(c0):
            c = slice(c0, c0 + hn)
            oc = slice(hn - c0, 2 * hn - c0)
            s2 = []
            for i in range(K):
                r = pl.ds(i * rs, rs)
                mk(PH_IN, i, r1_buf.at[r], r1_buf.at[r],
                   (my_x, end_y, my_z)).wait_recv()
                out_ref[r, c] = (
                    x_ref[r, c].astype(out_ref.dtype) + r1_buf[r]
                )
                rdma = mk(PH_MID, i, out_ref.at[r, c], r2_buf.at[r],
                          (my_x, om_y, my_z))
                rdma.start()
                s2.append(rdma)
            s3, s4 = [], []
            for i in range(K):
                r = pl.ds(i * rs, rs)
                mk(PH_MID, i, out_ref.at[r, c], r2_buf.at[r],
                   (my_x, om_y, my_z)).wait_recv()
                s2[i].wait_send()
                out_ref[r, c] = out_ref[r, c] + r2_buf[r]
                rdma = mk(PH_OUT, i, out_ref.at[r, c], out_ref.at[r, c],
                          (my_x, end_y, my_z))
                rdma.start()
                s3.append(rdma)
                rdma = mk(PH_X, i, out_ref.at[r, c], out_ref.at[r, c],
                          (xp, my_y, my_z))
                rdma.start()
                s4.append(rdma)
            for i in range(K):
                r = pl.ds(i * rs, rs)
                mk(PH_X, i, out_ref.at[r, c], out_ref.at[r, oc],
                   (xp, my_y, my_z)).wait_recv()
            for i in range(K):
                s3[i].wait_send()
                s4[i].wait_send()

        not_end = jnp.logical_not(is_end)

        @pl.when(jnp.logical_and(is_end, my_x == 0))
        def _():
            end_program(0)

        @pl.when(jnp.logical_and(is_end, my_x == 1))
        def _():
            end_program(hn)

        @pl.when(jnp.logical_and(not_end, my_x == 0))
        def _():
            mid_program(0)

        @pl.when(jnp.logical_and(not_end, my_x == 1))
        def _():
            mid_program(hn)

    return pl.pallas_call(
        body,
        out_shape=jax.ShapeDtypeStruct((m, n), jnp.bfloat16),
        in_specs=[pl.BlockSpec(memory_space=pltpu.VMEM)],
        out_specs=pl.BlockSpec(memory_space=pltpu.VMEM),
        scratch_shapes=[
            pltpu.VMEM((m, hn), jnp.bfloat16),
            pltpu.VMEM((m, hn), jnp.bfloat16),
            pltpu.SemaphoreType.DMA((4, K)),
            pltpu.SemaphoreType.DMA((4, K)),
        ],
        compiler_params=pltpu.CompilerParams(collective_id=0),
    )(x)


# device time: 99610 ns/iter; 1.6134x vs baseline; 1.1170x over previous
import jax
import jax.numpy as jnp
from jax import lax
from jax.experimental import pallas as pl
from jax.experimental.pallas import tpu as pltpu

P = 4
K = 16

PH_IN = 0
PH_MID = 1
PH_OUT = 2
PH_X = 3


def kernel(x):
    m, n = x.shape
    hn = n // 2
    rs = m // K

    def body(x_ref, out_ref, r1_buf, r2_buf, send_sems, recv_sems):
        my_x = lax.axis_index("x")
        my_y = lax.axis_index("y")
        my_z = lax.axis_index("z")

        is_end = jnp.logical_or(my_y == 0, my_y == 3)
        inner_y = jnp.where(my_y == 0, 1, 2)
        end_y = jnp.where(my_y == 1, 0, 3)
        om_y = jnp.where(my_y == 1, 2, 1)
        xp = 1 - my_x

        barrier = pltpu.get_barrier_semaphore()

        @pl.when(is_end)
        def _():
            for dev in ((my_x, inner_y, my_z), (xp, my_y, my_z)):
                pl.semaphore_signal(
                    barrier, inc=1, device_id=dev,
                    device_id_type=pl.DeviceIdType.MESH,
                )
            pl.semaphore_wait(barrier, 2)

        @pl.when(jnp.logical_not(is_end))
        def _():
            for dev in (
                (my_x, end_y, my_z),
                (my_x, om_y, my_z),
                (xp, my_y, my_z),
            ):
                pl.semaphore_signal(
                    barrier, inc=1, device_id=dev,
                    device_id_type=pl.DeviceIdType.MESH,
                )
            pl.semaphore_wait(barrier, 3)

        def mk(phase, i, src, dst, dev):
            return pltpu.make_async_remote_copy(
                src_ref=src,
                dst_ref=dst,
                send_sem=send_sems.at[phase, i],
                recv_sem=recv_sems.at[phase, i],
                device_id=dev,
                device_id_type=pl.DeviceIdType.MESH,
            )

        def end_program(c0):
            c = slice(c0, c0 + hn)
            oc = slice(hn - c0, 2 * hn - c0)
            s1 = []
            for i in range(K):
                r = pl.ds(i * rs, rs)
                out_ref[r, c] = x_ref[r, c].astype(out_ref.dtype)
                rdma = mk(PH_IN, i, out_ref.at[r, c], r1_buf.at[r],
                          (my_x, inner_y, my_z))
                rdma.start()
                s1.append(rdma)
            s4 = []
            for i in range(K):
                r = pl.ds(i * rs, rs)
                mk(PH_OUT, i, out_ref.at[r, c], out_ref.at[r, c],
                   (my_x, inner_y, my_z)).wait_recv()
                rdma = mk(PH_X, i, out_ref.at[r, c], out_ref.at[r, c],
                          (xp, my_y, my_z))
                rdma.start()
                s4.append(rdma)
            for i in range(K):
                r = pl.ds(i * rs, rs)
                mk(PH_X, i, out_ref.at[r, c], out_ref.at[r, oc],
                   (xp, my_y, my_z)).wait_recv()
            for i in range(K):
                s1[i].wait_send()
                s4[i].wait_send()

        def mid_program(c0):
            c = slice(c0, c0 + hn)
            oc = slice(hn - c0, 2 * hn - c0)
            s2, s3, s4 = [], [], []
            for i in range(K):
                r = pl.ds(i * rs, rs)
                mk(PH_IN, i, r1_buf.at[r], r1_buf.at[r],
                   (my_x, end_y, my_z)).wait_recv()
                out_ref[r, c] = (
                    x_ref[r, c].astype(out_ref.dtype) + r1_buf[r]
                )
                rdma = mk(PH_MID, i, out_ref.at[r, c], r2_buf.at[r],
                          (my_x, om_y, my_z))
                rdma.start()
                s2.append(rdma)
                mk(PH_MID, i, out_ref.at[r, c], r2_buf.at[r],
                   (my_x, om_y, my_z)).wait_recv()
                s2[i].wait_send()
                out_ref[r, c] = out_ref[r, c] + r2_buf[r]
                rdma = mk(PH_OUT, i, out_ref.at[r, c], out_ref.at[r, c],
                          (my_x, end_y, my_z))
                rdma.start()
                s3.append(rdma)
                rdma = mk(PH_X, i, out_ref.at[r, c], out_ref.at[r, c],
                          (xp, my_y, my_z))
                rdma.start()
                s4.append(rdma)
            for i in range(K):
                r = pl.ds(i * rs, rs)
                mk(PH_X, i, out_ref.at[r, c], out_ref.at[r, oc],
                   (xp, my_y, my_z)).wait_recv()
            for i in range(K):
                s3[i].wait_send()
                s4[i].wait_send()

        not_end = jnp.logical_not(is_end)

        @pl.when(jnp.logical_and(is_end, my_x == 0))
        def _():
            end_program(0)

        @pl.when(jnp.logical_and(is_end, my_x == 1))
        def _():
            end_program(hn)

        @pl.when(jnp.logical_and(not_end, my_x == 0))
        def _():
            mid_program(0)

        @pl.when(jnp.logical_and(not_end, my_x == 1))
        def _():
            mid_program(hn)

    return pl.pallas_call(
        body,
        out_shape=jax.ShapeDtypeStruct((m, n), jnp.bfloat16),
        in_specs=[pl.BlockSpec(memory_space=pltpu.VMEM)],
        out_specs=pl.BlockSpec(memory_space=pltpu.VMEM),
        scratch_shapes=[
            pltpu.VMEM((m, hn), jnp.bfloat16),
            pltpu.VMEM((m, hn), jnp.bfloat16),
            pltpu.SemaphoreType.DMA((4, K)),
            pltpu.SemaphoreType.DMA((4, K)),
        ],
        compiler_params=pltpu.CompilerParams(collective_id=0),
    )(x)


# device time: 99490 ns/iter; 1.6153x vs baseline; 1.0012x over previous
import jax
import jax.numpy as jnp
from jax import lax
from jax.experimental import pallas as pl
from jax.experimental.pallas import tpu as pltpu

P = 4
K = 16

PH_IN = 0
PH_MID = 1
PH_OUT = 2
PH_X = 3


def kernel(x):
    m, n = x.shape
    mh = m // 2
    rs = mh // K

    def body(x_ref, out_ref, r1_buf, r2_buf, send_sems, recv_sems):
        my_x = lax.axis_index("x")
        my_y = lax.axis_index("y")
        my_z = lax.axis_index("z")

        is_end = jnp.logical_or(my_y == 0, my_y == 3)
        inner_y = jnp.where(my_y == 0, 1, 2)
        end_y = jnp.where(my_y == 1, 0, 3)
        om_y = jnp.where(my_y == 1, 2, 1)
        xp = 1 - my_x

        barrier = pltpu.get_barrier_semaphore()

        @pl.when(is_end)
        def _():
            for dev in ((my_x, inner_y, my_z), (xp, my_y, my_z)):
                pl.semaphore_signal(
                    barrier, inc=1, device_id=dev,
                    device_id_type=pl.DeviceIdType.MESH,
                )
            pl.semaphore_wait(barrier, 2)

        @pl.when(jnp.logical_not(is_end))
        def _():
            for dev in (
                (my_x, end_y, my_z),
                (my_x, om_y, my_z),
                (xp, my_y, my_z),
            ):
                pl.semaphore_signal(
                    barrier, inc=1, device_id=dev,
                    device_id_type=pl.DeviceIdType.MESH,
                )
            pl.semaphore_wait(barrier, 3)

        def mk(phase, i, src, dst, dev):
            return pltpu.make_async_remote_copy(
                src_ref=src,
                dst_ref=dst,
                send_sem=send_sems.at[phase, i],
                recv_sem=recv_sems.at[phase, i],
                device_id=dev,
                device_id_type=pl.DeviceIdType.MESH,
            )

        r0 = my_x * mh
        o0 = mh - r0

        def end_program():
            s1 = []
            for i in range(K):
                g = pl.ds(r0 + i * rs, rs)
                l = pl.ds(i * rs, rs)
                out_ref[g, :] = x_ref[g, :].astype(out_ref.dtype)
                rdma = mk(PH_IN, i, out_ref.at[g], r1_buf.at[l],
                          (my_x, inner_y, my_z))
                rdma.start()
                s1.append(rdma)
            s4 = []
            for i in range(K):
                g = pl.ds(r0 + i * rs, rs)
                mk(PH_OUT, i, out_ref.at[g], out_ref.at[g],
                   (my_x, inner_y, my_z)).wait_recv()
                rdma = mk(PH_X, i, out_ref.at[g], out_ref.at[g],
                          (xp, my_y, my_z))
                rdma.start()
                s4.append(rdma)
            for i in range(K):
                og = pl.ds(o0 + i * rs, rs)
                mk(PH_X, i, out_ref.at[og], out_ref.at[og],
                   (xp, my_y, my_z)).wait_recv()
            for i in range(K):
                s1[i].wait_send()
                s4[i].wait_send()

        def mid_program():
            s2, s3, s4 = [], [], []
            for i in range(K):
                g = pl.ds(r0 + i * rs, rs)
                l = pl.ds(i * rs, rs)
                mk(PH_IN, i, r1_buf.at[l], r1_buf.at[l],
                   (my_x, end_y, my_z)).wait_recv()
                out_ref[g, :] = (
                    x_ref[g, :].astype(out_ref.dtype) + r1_buf[l, :]
                )
                rdma = mk(PH_MID, i, out_ref.at[g], r2_buf.at[l],
                          (my_x, om_y, my_z))
                rdma.start()
                s2.append(rdma)
                mk(PH_MID, i, out_ref.at[g], r2_buf.at[l],
                   (my_x, om_y, my_z)).wait_recv()
                s2[i].wait_send()
                out_ref[g, :] = out_ref[g, :] + r2_buf[l, :]
                rdma = mk(PH_OUT, i, out_ref.at[g], out_ref.at[g],
                          (my_x, end_y, my_z))
                rdma.start()
                s3.append(rdma)
                rdma = mk(PH_X, i, out_ref.at[g], out_ref.at[g],
                          (xp, my_y, my_z))
                rdma.start()
                s4.append(rdma)
            for i in range(K):
                og = pl.ds(o0 + i * rs, rs)
                mk(PH_X, i, out_ref.at[og], out_ref.at[og],
                   (xp, my_y, my_z)).wait_recv()
            for i in range(K):
                s3[i].wait_send()
                s4[i].wait_send()

        @pl.when(is_end)
        def _():
            end_program()

        @pl.when(jnp.logical_not(is_end))
        def _():
            mid_program()

    return pl.pallas_call(
        body,
        out_shape=jax.ShapeDtypeStruct((m, n), jnp.bfloat16),
        in_specs=[pl.BlockSpec(memory_space=pltpu.VMEM)],
        out_specs=pl.BlockSpec(memory_space=pltpu.VMEM),
        scratch_shapes=[
            pltpu.VMEM((m // 2, n), jnp.bfloat16),
            pltpu.VMEM((m // 2, n), jnp.bfloat16),
            pltpu.SemaphoreType.DMA((4, K)),
            pltpu.SemaphoreType.DMA((4, K)),
        ],
        compiler_params=pltpu.CompilerParams(collective_id=0),
    )(x)
